# baseline (speedup 1.0000x reference)
import functools

import jax
import jax.numpy as jnp
from jax import lax
from jax.experimental import pallas as pl
from jax.experimental.pallas import tpu as pltpu
from jax.experimental.pallas import tpu_sc as plsc

B = 1024
DF = 128
M = 100000
MP = 100352
TM = 1024
NUM_CLASSES = 1000
CONF_THRESHOLD = 0.8
EPS = 1e-8
BIGIDX = 2**30


def _sim_body(x_ref, wbb_ref, bbb_ref, keys_ref,
              feat_ref, conf_ref, idx_ref, fn_ref):
    m = pl.program_id(0)

    @pl.when(m == 0)
    def _init():
        f = jnp.dot(x_ref[...], wbb_ref[...],
                    preferred_element_type=jnp.float32) + bbb_ref[...]
        feat_ref[...] = f
        n = jnp.sqrt(jnp.sum(f * f, axis=1, keepdims=True))
        fn_ref[...] = f / (n + EPS)
        conf_ref[...] = jnp.full((B, 1), -jnp.inf, jnp.float32)
        idx_ref[...] = jnp.zeros((B, 1), jnp.int32)

    k = keys_ref[...]
    kn = jnp.sqrt(jnp.sum(k * k, axis=1, keepdims=True))
    k = k / (kn + EPS)
    sim = lax.dot_general(fn_ref[...], k, (((1,), (1,)), ((), ())),
                          preferred_element_type=jnp.float32)
    gcol = m * TM + lax.broadcasted_iota(jnp.int32, (B, TM), 1)
    lmax = jnp.max(sim, axis=1, keepdims=True)
    lidx = jnp.min(jnp.where(sim == lmax, gcol, BIGIDX),
                   axis=1, keepdims=True)
    better = lmax > conf_ref[...]
    conf_ref[...] = jnp.where(better, lmax, conf_ref[...])
    idx_ref[...] = jnp.where(better, lidx, idx_ref[...])


def _run_sim(x, W_bb, b_bb, keys_pad):
    return pl.pallas_call(
        _sim_body,
        grid=(MP // TM,),
        in_specs=[
            pl.BlockSpec((B, 768), lambda m: (0, 0)),
            pl.BlockSpec((768, DF), lambda m: (0, 0)),
            pl.BlockSpec((1, DF), lambda m: (0, 0)),
            pl.BlockSpec((TM, DF), lambda m: (m, 0)),
        ],
        out_specs=[
            pl.BlockSpec((B, DF), lambda m: (0, 0)),
            pl.BlockSpec((B, 1), lambda m: (0, 0)),
            pl.BlockSpec((B, 1), lambda m: (0, 0)),
        ],
        out_shape=[
            jax.ShapeDtypeStruct((B, DF), jnp.float32),
            jax.ShapeDtypeStruct((B, 1), jnp.float32),
            jax.ShapeDtypeStruct((B, 1), jnp.int32),
        ],
        scratch_shapes=[pltpu.VMEM((B, DF), jnp.float32)],
        compiler_params=pltpu.CompilerParams(
            dimension_semantics=("arbitrary",)),
    )(x, W_bb, b_bb.reshape(1, DF), keys_pad)


@functools.cache
def _make_sc_gather():
    info = plsc.get_sparse_core_info()
    nw = info.num_cores * info.num_subcores
    bpw = B // nw
    mesh = plsc.VectorSubcoreMesh(core_axis_name="c", subcore_axis_name="s")

    @functools.partial(
        pl.kernel, mesh=mesh,
        out_type=jax.ShapeDtypeStruct((B,), jnp.int32),
        scratch_types=[
            pltpu.VMEM((bpw,), jnp.int32),
            pltpu.VMEM((bpw,), jnp.int32),
            pltpu.SemaphoreType.DMA,
        ],
    )
    def gather(labels_hbm, idx_hbm, out_hbm, idx_v, rows_v, sem):
        wid = lax.axis_index("s") * info.num_cores + lax.axis_index("c")
        base = wid * bpw
        pltpu.sync_copy(idx_hbm.at[pl.ds(base, bpw)], idx_v)
        pltpu.async_copy(labels_hbm.at[idx_v], rows_v, sem).wait()
        pltpu.sync_copy(rows_v, out_hbm.at[pl.ds(base, bpw)])

    return gather


def _head_body(feat_ref, wcls_ref, bcls_ref, labels_ref, conf_ref,
               out_ref, src_ref):
    logits = jnp.dot(feat_ref[...], wcls_ref[...],
                     preferred_element_type=jnp.float32) + bcls_ref[...]
    cls = lax.broadcasted_iota(jnp.int32, (B, NUM_CLASSES), 1)
    onehot = (labels_ref[...] == cls).astype(jnp.float32)
    use_mem = conf_ref[...] >= CONF_THRESHOLD
    out_ref[...] = jnp.where(use_mem, onehot, logits)
    src_ref[...] = use_mem.astype(jnp.int32)


def _run_head(features, W_cls, b_cls, labels, conf):
    return pl.pallas_call(
        _head_body,
        out_shape=[
            jax.ShapeDtypeStruct((B, NUM_CLASSES), jnp.float32),
            jax.ShapeDtypeStruct((B, 1), jnp.int32),
        ],
    )(features, W_cls, b_cls.reshape(1, NUM_CLASSES),
      labels.reshape(B, 1), conf)


def kernel(x, W_bb, b_bb, W_cls, b_cls, memory_keys, memory_labels):
    keys_pad = jnp.pad(memory_keys, ((0, MP - M), (0, 0)), mode="edge")
    features, conf, idx = _run_sim(x, W_bb, b_bb, keys_pad)
    labels = _make_sc_gather()(memory_labels.astype(jnp.int32), idx.reshape(B))
    final_logits, sources = _run_head(features, W_cls, b_cls, labels, conf)
    return final_logits, sources.reshape(B)


# speedup vs baseline: 1.3079x; 1.3079x over previous
import functools

import jax
import jax.numpy as jnp
from jax import lax
from jax.experimental import pallas as pl
from jax.experimental.pallas import tpu as pltpu
from jax.experimental.pallas import tpu_sc as plsc

B = 1024
DF = 128
M = 100000
TM = 2000
NUM_CLASSES = 1000
CONF_THRESHOLD = 0.8
EPS = 1e-8
BIGIDX = 2**30


def _sim_body(x_ref, wbb_ref, bbb_ref, keys_ref,
              feat_ref, conf_ref, idx_ref, fn_ref):
    m = pl.program_id(0)

    @pl.when(m == 0)
    def _init():
        f = jnp.dot(x_ref[...], wbb_ref[...],
                    preferred_element_type=jnp.float32) + bbb_ref[...]
        feat_ref[...] = f
        n = jnp.sqrt(jnp.sum(f * f, axis=1, keepdims=True))
        fn_ref[...] = f / (n + EPS)
        conf_ref[...] = jnp.full((B, 1), -jnp.inf, jnp.float32)
        idx_ref[...] = jnp.zeros((B, 1), jnp.int32)

    k = keys_ref[...]
    kn = jnp.sqrt(jnp.sum(k * k, axis=1, keepdims=True))
    k = (k / (kn + EPS)).astype(jnp.bfloat16)
    fnb = fn_ref[...].astype(jnp.bfloat16)
    sim = lax.dot_general(fnb, k, (((1,), (1,)), ((), ())),
                          preferred_element_type=jnp.float32)
    liota = lax.broadcasted_iota(jnp.int32, (1, TM), 1)
    lmax = jnp.max(sim, axis=1, keepdims=True)
    lidx = m * TM + jnp.min(jnp.where(sim == lmax, liota, BIGIDX),
                            axis=1, keepdims=True)
    better = lmax > conf_ref[...]
    conf_ref[...] = jnp.where(better, lmax, conf_ref[...])
    idx_ref[...] = jnp.where(better, lidx, idx_ref[...])


def _run_sim(x, W_bb, b_bb, keys):
    return pl.pallas_call(
        _sim_body,
        grid=(M // TM,),
        in_specs=[
            pl.BlockSpec((B, 768), lambda m: (0, 0)),
            pl.BlockSpec((768, DF), lambda m: (0, 0)),
            pl.BlockSpec((1, DF), lambda m: (0, 0)),
            pl.BlockSpec((TM, DF), lambda m: (m, 0)),
        ],
        out_specs=[
            pl.BlockSpec((B, DF), lambda m: (0, 0)),
            pl.BlockSpec((B, 1), lambda m: (0, 0)),
            pl.BlockSpec((B, 1), lambda m: (0, 0)),
        ],
        out_shape=[
            jax.ShapeDtypeStruct((B, DF), jnp.float32),
            jax.ShapeDtypeStruct((B, 1), jnp.float32),
            jax.ShapeDtypeStruct((B, 1), jnp.int32),
        ],
        scratch_shapes=[pltpu.VMEM((B, DF), jnp.float32)],
        compiler_params=pltpu.CompilerParams(
            dimension_semantics=("arbitrary",)),
    )(x, W_bb, b_bb.reshape(1, DF), keys)


@functools.cache
def _make_sc_gather():
    info = plsc.get_sparse_core_info()
    nw = info.num_cores * info.num_subcores
    bpw = B // nw
    mesh = plsc.VectorSubcoreMesh(core_axis_name="c", subcore_axis_name="s")

    @functools.partial(
        pl.kernel, mesh=mesh,
        out_type=jax.ShapeDtypeStruct((B,), jnp.int32),
        scratch_types=[
            pltpu.VMEM((bpw,), jnp.int32),
            pltpu.VMEM((bpw,), jnp.int32),
            pltpu.SemaphoreType.DMA,
        ],
    )
    def gather(labels_hbm, idx_hbm, out_hbm, idx_v, rows_v, sem):
        wid = lax.axis_index("s") * info.num_cores + lax.axis_index("c")
        base = wid * bpw
        pltpu.sync_copy(idx_hbm.at[pl.ds(base, bpw)], idx_v)
        pltpu.async_copy(labels_hbm.at[idx_v], rows_v, sem).wait()
        pltpu.sync_copy(rows_v, out_hbm.at[pl.ds(base, bpw)])

    return gather


def _head_body(feat_ref, wcls_ref, bcls_ref, labels_ref, conf_ref,
               out_ref, src_ref):
    logits = jnp.dot(feat_ref[...], wcls_ref[...],
                     preferred_element_type=jnp.float32) + bcls_ref[...]
    cls = lax.broadcasted_iota(jnp.int32, (B, NUM_CLASSES), 1)
    onehot = (labels_ref[...] == cls).astype(jnp.float32)
    use_mem = conf_ref[...] >= CONF_THRESHOLD
    out_ref[...] = jnp.where(use_mem, onehot, logits)
    src_ref[...] = use_mem.astype(jnp.int32)


def _run_head(features, W_cls, b_cls, labels, conf):
    return pl.pallas_call(
        _head_body,
        out_shape=[
            jax.ShapeDtypeStruct((B, NUM_CLASSES), jnp.float32),
            jax.ShapeDtypeStruct((B, 1), jnp.int32),
        ],
    )(features, W_cls, b_cls.reshape(1, NUM_CLASSES),
      labels.reshape(B, 1), conf)


def kernel(x, W_bb, b_bb, W_cls, b_cls, memory_keys, memory_labels):
    features, conf, idx = _run_sim(x, W_bb, b_bb, memory_keys)
    labels = _make_sc_gather()(memory_labels.astype(jnp.int32), idx.reshape(B))
    final_logits, sources = _run_head(features, W_cls, b_cls, labels, conf)
    return final_logits, sources.reshape(B)
